# baseline (speedup 1.0000x reference)
import functools
import math

import jax
import jax.numpy as jnp
from jax.experimental import pallas as pl
from jax.experimental.pallas import tpu as pltpu

_TB = 128


def _top1_kernel(x_ref, wg_ref, laux_ref, comb_ref, disp_ref,
                 counts_ref, gsum_ref, *, num_tokens, num_experts, capacity):
    i = pl.program_id(0)
    nb = pl.num_programs(0)

    @pl.when(i == 0)
    def _init():
        counts_ref[...] = jnp.zeros_like(counts_ref)
        gsum_ref[...] = jnp.zeros_like(gsum_ref)

    xb = x_ref[...]
    wgt = wg_ref[...]
    logits = jax.lax.dot_general(
        xb, wgt, (((1,), (1,)), ((), ())),
        preferred_element_type=jnp.float32)

    m = jnp.max(logits, axis=1, keepdims=True)
    eg = jnp.exp(logits - m)
    sg = jnp.sum(eg, axis=1, keepdims=True)
    gates = eg / sg

    gmax = jnp.max(gates, axis=1, keepdims=True)
    iota_e = jax.lax.broadcasted_iota(jnp.int32, (_TB, num_experts), 1)
    idx = jnp.min(jnp.where(gates == gmax, iota_e, num_experts),
                  axis=1, keepdims=True)
    mask1 = (iota_e == idx).astype(jnp.float32)

    r = jax.lax.broadcasted_iota(jnp.int32, (_TB, _TB), 0)
    c = jax.lax.broadcasted_iota(jnp.int32, (_TB, _TB), 1)
    tri = (r >= c).astype(jnp.float32)
    incl = jax.lax.dot_general(
        tri, mask1, (((1,), (0,)), ((), ())),
        preferred_element_type=jnp.float32)

    base = counts_ref[...]
    loc = incl + base - 1.0
    keep = mask1 * (loc < capacity).astype(jnp.float32)
    gates1 = gmax * keep
    loc_s = jnp.sum(loc * keep, axis=1).astype(jnp.int32)
    iota_c = jax.lax.broadcasted_iota(jnp.int32, (_TB, capacity), 1)
    row_c = (iota_c == loc_s[:, None]).astype(jnp.float32)

    comb = gates1[:, :, None] * row_c[:, None, :]
    comb_ref[...] = comb
    disp_ref[...] = comb != 0.0

    counts_ref[...] = base + jnp.sum(mask1, axis=0, keepdims=True)
    gsum_ref[...] = gsum_ref[...] + jnp.sum(gates, axis=0, keepdims=True)

    @pl.when(i == nb - 1)
    def _finish():
        s = jnp.sum(gsum_ref[...] * counts_ref[...])
        laux_ref[...] = jnp.full((1, 1), num_experts, jnp.float32) * s \
            / (num_tokens * num_tokens)


def kernel(x, wg):
    num_tokens, model_dim = x.shape
    num_experts = wg.shape[0]
    capacity = int(math.ceil(num_tokens / num_experts))
    nb = num_tokens // _TB

    kfn = functools.partial(
        _top1_kernel, num_tokens=num_tokens, num_experts=num_experts,
        capacity=capacity)

    laux, comb, disp = pl.pallas_call(
        kfn,
        grid=(nb,),
        in_specs=[
            pl.BlockSpec((_TB, model_dim), lambda i: (i, 0)),
            pl.BlockSpec((num_experts, model_dim), lambda i: (0, 0)),
        ],
        out_specs=[
            pl.BlockSpec((1, 1), lambda i: (0, 0)),
            pl.BlockSpec((_TB, num_experts, capacity), lambda i: (i, 0, 0)),
            pl.BlockSpec((_TB, num_experts, capacity), lambda i: (i, 0, 0)),
        ],
        out_shape=[
            jax.ShapeDtypeStruct((1, 1), jnp.float32),
            jax.ShapeDtypeStruct((num_tokens, num_experts, capacity),
                                 jnp.float32),
            jax.ShapeDtypeStruct((num_tokens, num_experts, capacity),
                                 jnp.bool_),
        ],
        scratch_shapes=[
            pltpu.VMEM((1, num_experts), jnp.float32),
            pltpu.VMEM((1, num_experts), jnp.float32),
        ],
    )(x, wg)
    return laux.reshape(()), comb, disp


# speedup vs baseline: 1.0138x; 1.0138x over previous
import functools
import math

import jax
import jax.numpy as jnp
from jax.experimental import pallas as pl
from jax.experimental.pallas import tpu as pltpu

_TB = 128


def _top1_kernel(x_ref, wg_ref, laux_ref, comb_ref, disp_ref,
                 counts_ref, gsum_ref, *, num_tokens, num_experts, capacity):
    i = pl.program_id(0)
    nb = pl.num_programs(0)

    @pl.when(i == 0)
    def _init():
        counts_ref[...] = jnp.zeros_like(counts_ref)
        gsum_ref[...] = jnp.zeros_like(gsum_ref)

    xb = x_ref[...]
    wgt = wg_ref[...]
    logits = jax.lax.dot_general(
        xb, wgt, (((1,), (1,)), ((), ())),
        preferred_element_type=jnp.float32)

    m = jnp.max(logits, axis=1, keepdims=True)
    eg = jnp.exp(logits - m)
    sg = jnp.sum(eg, axis=1, keepdims=True)
    gates = eg / sg

    gmax = jnp.max(gates, axis=1, keepdims=True)
    iota_e = jax.lax.broadcasted_iota(jnp.int32, (_TB, num_experts), 1)
    idx = jnp.min(jnp.where(gates == gmax, iota_e, num_experts),
                  axis=1, keepdims=True)
    mask1 = (iota_e == idx).astype(jnp.float32)

    r = jax.lax.broadcasted_iota(jnp.int32, (_TB, _TB), 0)
    c = jax.lax.broadcasted_iota(jnp.int32, (_TB, _TB), 1)
    tri = (r >= c).astype(jnp.float32)
    incl = jax.lax.dot_general(
        tri, mask1, (((1,), (0,)), ((), ())),
        preferred_element_type=jnp.float32)

    base = counts_ref[...]
    loc_tok = jnp.sum((incl + base) * mask1, axis=1, keepdims=True) - 1.0
    kept = loc_tok < capacity
    code = jnp.where(kept, idx * capacity + loc_tok.astype(jnp.int32),
                     -1)
    iota3 = (jax.lax.broadcasted_iota(jnp.int32, (_TB, num_experts, capacity), 1)
             * capacity
             + jax.lax.broadcasted_iota(jnp.int32, (_TB, num_experts, capacity), 2))
    cond = iota3 == code[:, :, None]
    comb_ref[...] = jnp.where(cond, gmax[:, :, None], 0.0)
    disp_ref[...] = cond

    counts_ref[...] = base + jnp.sum(mask1, axis=0, keepdims=True)
    gsum_ref[...] = gsum_ref[...] + jnp.sum(gates, axis=0, keepdims=True)

    @pl.when(i == nb - 1)
    def _finish():
        s = jnp.sum(gsum_ref[...] * counts_ref[...])
        laux_ref[...] = jnp.full((1, 1), num_experts, jnp.float32) * s \
            / (num_tokens * num_tokens)


def kernel(x, wg):
    num_tokens, model_dim = x.shape
    num_experts = wg.shape[0]
    capacity = int(math.ceil(num_tokens / num_experts))
    nb = num_tokens // _TB

    kfn = functools.partial(
        _top1_kernel, num_tokens=num_tokens, num_experts=num_experts,
        capacity=capacity)

    laux, comb, disp = pl.pallas_call(
        kfn,
        grid=(nb,),
        in_specs=[
            pl.BlockSpec((_TB, model_dim), lambda i: (i, 0)),
            pl.BlockSpec((num_experts, model_dim), lambda i: (0, 0)),
        ],
        out_specs=[
            pl.BlockSpec((1, 1), lambda i: (0, 0)),
            pl.BlockSpec((_TB, num_experts, capacity), lambda i: (i, 0, 0)),
            pl.BlockSpec((_TB, num_experts, capacity), lambda i: (i, 0, 0)),
        ],
        out_shape=[
            jax.ShapeDtypeStruct((1, 1), jnp.float32),
            jax.ShapeDtypeStruct((num_tokens, num_experts, capacity),
                                 jnp.float32),
            jax.ShapeDtypeStruct((num_tokens, num_experts, capacity),
                                 jnp.bool_),
        ],
        scratch_shapes=[
            pltpu.VMEM((1, num_experts), jnp.float32),
            pltpu.VMEM((1, num_experts), jnp.float32),
        ],
    )(x, wg)
    return laux.reshape(()), comb, disp


# speedup vs baseline: 1.0422x; 1.0280x over previous
import functools
import math

import jax
import jax.numpy as jnp
from jax.experimental import pallas as pl
from jax.experimental.pallas import tpu as pltpu

_TB = 256


def _top1_kernel(x_ref, wg_ref, laux_ref, comb_ref, disp_ref,
                 counts_ref, gsum_ref, *, num_tokens, num_experts, capacity):
    i = pl.program_id(0)
    nb = pl.num_programs(0)

    @pl.when(i == 0)
    def _init():
        counts_ref[...] = jnp.zeros_like(counts_ref)
        gsum_ref[...] = jnp.zeros_like(gsum_ref)

    xb = x_ref[...]
    wgt = wg_ref[...]
    logits = jax.lax.dot_general(
        xb, wgt, (((1,), (1,)), ((), ())),
        preferred_element_type=jnp.float32)

    m = jnp.max(logits, axis=1, keepdims=True)
    eg = jnp.exp(logits - m)
    sg = jnp.sum(eg, axis=1, keepdims=True)
    gates = eg / sg

    gmax = jnp.max(gates, axis=1, keepdims=True)
    iota_e = jax.lax.broadcasted_iota(jnp.int32, (_TB, num_experts), 1)
    idx = jnp.min(jnp.where(gates == gmax, iota_e, num_experts),
                  axis=1, keepdims=True)
    mask1 = (iota_e == idx).astype(jnp.float32)

    r = jax.lax.broadcasted_iota(jnp.int32, (_TB, _TB), 0)
    c = jax.lax.broadcasted_iota(jnp.int32, (_TB, _TB), 1)
    tri = (r >= c).astype(jnp.float32)
    incl = jax.lax.dot_general(
        tri, mask1, (((1,), (0,)), ((), ())),
        preferred_element_type=jnp.float32)

    base = counts_ref[...]
    loc_tok = jnp.sum((incl + base) * mask1, axis=1, keepdims=True) - 1.0
    kept = loc_tok < capacity
    code = jnp.where(kept, idx * capacity + loc_tok.astype(jnp.int32),
                     -1)
    iota3 = (jax.lax.broadcasted_iota(jnp.int32, (_TB, num_experts, capacity), 1)
             * capacity
             + jax.lax.broadcasted_iota(jnp.int32, (_TB, num_experts, capacity), 2))
    cond = iota3 == code[:, :, None]
    comb_ref[...] = jnp.where(cond, gmax[:, :, None], 0.0)
    disp_ref[...] = cond

    counts_ref[...] = base + jnp.sum(mask1, axis=0, keepdims=True)
    gsum_ref[...] = gsum_ref[...] + jnp.sum(gates, axis=0, keepdims=True)

    @pl.when(i == nb - 1)
    def _finish():
        s = jnp.sum(gsum_ref[...] * counts_ref[...])
        laux_ref[...] = jnp.full((1, 1), num_experts, jnp.float32) * s \
            / (num_tokens * num_tokens)


def kernel(x, wg):
    num_tokens, model_dim = x.shape
    num_experts = wg.shape[0]
    capacity = int(math.ceil(num_tokens / num_experts))
    nb = num_tokens // _TB

    kfn = functools.partial(
        _top1_kernel, num_tokens=num_tokens, num_experts=num_experts,
        capacity=capacity)

    laux, comb, disp = pl.pallas_call(
        kfn,
        grid=(nb,),
        in_specs=[
            pl.BlockSpec((_TB, model_dim), lambda i: (i, 0)),
            pl.BlockSpec((num_experts, model_dim), lambda i: (0, 0)),
        ],
        out_specs=[
            pl.BlockSpec((1, 1), lambda i: (0, 0)),
            pl.BlockSpec((_TB, num_experts, capacity), lambda i: (i, 0, 0)),
            pl.BlockSpec((_TB, num_experts, capacity), lambda i: (i, 0, 0)),
        ],
        out_shape=[
            jax.ShapeDtypeStruct((1, 1), jnp.float32),
            jax.ShapeDtypeStruct((num_tokens, num_experts, capacity),
                                 jnp.float32),
            jax.ShapeDtypeStruct((num_tokens, num_experts, capacity),
                                 jnp.bool_),
        ],
        scratch_shapes=[
            pltpu.VMEM((1, num_experts), jnp.float32),
            pltpu.VMEM((1, num_experts), jnp.float32),
        ],
    )(x, wg)
    return laux.reshape(()), comb, disp
